# Initial kernel scaffold; baseline (speedup 1.0000x reference)
import jax
import jax.numpy as jnp
from jax.experimental import pallas as pl


def kernel(x, ref):
    raise NotImplementedError("write your pallas kernel here")


# speedup vs baseline: 14.3618x; 14.3618x over previous
import functools

import jax
import jax.numpy as jnp
from jax.experimental import pallas as pl

K = 4
P = 128
HO = 63
L = HO * HO
LP = 4096
TL = 256
NEG = -3.0e38


def _maxpool3(x):
    return jax.lax.reduce_window(x, -jnp.inf, jax.lax.max,
                                 (1, 1, 3, 3), (1, 1, 1, 1),
                                 [(0, 0), (0, 0), (1, 1), (1, 1)])


def _unfold(img):
    B, _, H, W = img.shape
    Ho = (H - 3) // 2 + 1
    Wo = (W - 3) // 2 + 1
    cols = []
    for di in range(3):
        for dj in range(3):
            cols.append(img[:, :, di:di + 2 * (Ho - 1) + 1:2,
                            dj:dj + 2 * (Wo - 1) + 1:2])
    patch = jnp.concatenate(cols, axis=1)
    return patch.reshape(B, 9, Ho * Wo)


def _ncc_topk_gather_body(inT_ref, refn_ref, refpT_ref, out_ref):
    a = inT_ref[0]
    bm = refn_ref[0]
    refpT = refpT_ref[0]

    ncc = jax.lax.dot_general(
        a, bm, (((1,), (0,)), ((), ())),
        preferred_element_type=jnp.float32,
        precision=jax.lax.Precision.DEFAULT)

    col = jax.lax.broadcasted_iota(jnp.int32, (TL, LP), 1)
    ncc = jnp.where(col < L, ncc, NEG)

    for i in range(K):
        m = jnp.max(ncc, axis=1, keepdims=True)
        idx = jnp.min(jnp.where(ncc == m, col, LP), axis=1)
        oh = col == idx[:, None]
        gathered = jax.lax.dot_general(
            oh.astype(jnp.float32), refpT, (((1,), (0,)), ((), ())),
            preferred_element_type=jnp.float32,
            precision=jax.lax.Precision.HIGHEST)
        out_ref[0, i] = gathered
        if i + 1 < K:
            ncc = jnp.where(oh, NEG, ncc)


def _fold_planes_body(g_ref, out_ref):
    g = g_ref[0]

    z_row = jnp.zeros((1, 64), jnp.float32)
    z_col = jnp.zeros((64, 1), jnp.float32)

    def shr_row(x):
        return jnp.concatenate([z_row, x[:63, :]], axis=0)

    def shr_col(x):
        return jnp.concatenate([z_col, x[:, :63]], axis=1)

    out_ref[0, 0, 0] = g[0] + shr_col(g[2]) + shr_row(g[6]) + shr_row(shr_col(g[8]))
    out_ref[0, 0, 1] = g[1] + shr_row(g[7])
    out_ref[0, 1, 0] = g[3] + shr_col(g[5])
    out_ref[0, 1, 1] = g[4]


@jax.jit
def kernel(x, ref):
    B = x.shape[0]
    x_gray = (jnp.max(_maxpool3(x), axis=1, keepdims=True)
              - jnp.min(x, axis=1, keepdims=True))
    ref_gray = jnp.mean(ref, axis=1, keepdims=True)
    input_patch = _unfold(x_gray)
    ref_patch = _unfold(ref_gray)

    input_mu = jnp.mean(input_patch, axis=2, keepdims=True)
    ref_mu = jnp.mean(ref_patch, axis=2, keepdims=True)
    input_len = jnp.linalg.norm(input_patch, axis=1, keepdims=True)
    ref_len = jnp.linalg.norm(ref_patch, axis=1, keepdims=True)
    input_norm = (input_patch - input_mu) / input_len
    ref_norm = (ref_patch - ref_mu) / ref_len

    pad9 = [(0, 0), (0, 16 - 9), (0, LP - L)]
    inT = jnp.pad(input_norm, pad9).transpose(0, 2, 1)
    refn = jnp.pad(ref_norm, pad9)
    refpT = jnp.pad(ref_patch, pad9).transpose(0, 2, 1)

    nblk = LP // TL
    gathered = pl.pallas_call(
        _ncc_topk_gather_body,
        grid=(B, nblk),
        in_specs=[
            pl.BlockSpec((1, TL, 16), lambda b, r: (b, r, 0)),
            pl.BlockSpec((1, 16, LP), lambda b, r: (b, 0, 0)),
            pl.BlockSpec((1, LP, 16), lambda b, r: (b, 0, 0)),
        ],
        out_specs=pl.BlockSpec((1, K, TL, 16), lambda b, r: (b, 0, r, 0)),
        out_shape=jax.ShapeDtypeStruct((B, K, LP, 16), jnp.float32),
    )(inT, refn, refpT)

    g = gathered[:, :, :L, :9].transpose(0, 1, 3, 2)
    g = g.reshape(B * K, 9, HO, HO)
    g = jnp.pad(g, [(0, 0), (0, 0), (0, 1), (0, 1)])

    planes = pl.pallas_call(
        _fold_planes_body,
        grid=(B * K,),
        in_specs=[pl.BlockSpec((1, 9, 64, 64), lambda i: (i, 0, 0, 0))],
        out_specs=pl.BlockSpec((1, 2, 2, 64, 64), lambda i: (i, 0, 0, 0, 0)),
        out_shape=jax.ShapeDtypeStruct((B * K, 2, 2, 64, 64), jnp.float32),
    )(g)

    folds = planes.reshape(B, K, 2, 2, 64, 64).transpose(0, 1, 4, 2, 5, 3)
    folds = folds.reshape(B, K, P, P)
    return jnp.concatenate([x, folds], axis=1)


# speedup vs baseline: 34.3989x; 2.3952x over previous
import functools

import jax
import jax.numpy as jnp
from jax import lax
from jax.experimental import pallas as pl
from jax.experimental.pallas import tpu as pltpu, tpu_sc as plsc

K = 4
P = 128
HO = 63
L = HO * HO
LP = 4096
TL = 256
NEG = -3.0e38

NW = 32
NCH = 16


def _maxpool3(x):
    return jax.lax.reduce_window(x, -jnp.inf, jax.lax.max,
                                 (1, 1, 3, 3), (1, 1, 1, 1),
                                 [(0, 0), (0, 0), (1, 1), (1, 1)])


def _unfold(img):
    B, _, H, W = img.shape
    Ho = (H - 3) // 2 + 1
    Wo = (W - 3) // 2 + 1
    cols = []
    for di in range(3):
        for dj in range(3):
            cols.append(img[:, :, di:di + 2 * (Ho - 1) + 1:2,
                            dj:dj + 2 * (Wo - 1) + 1:2])
    patch = jnp.concatenate(cols, axis=1)
    return patch.reshape(B, 9, Ho * Wo)


def _ncc_topk_body(inT_ref, refn_ref, idx_ref):
    a = inT_ref[0]
    bm = refn_ref[0]
    base = pl.program_id(0) * LP

    ncc = jax.lax.dot_general(
        a, bm, (((1,), (0,)), ((), ())),
        preferred_element_type=jnp.float32,
        precision=jax.lax.Precision.DEFAULT)

    col = jax.lax.broadcasted_iota(jnp.int32, (TL, LP), 1)
    ncc = jnp.where(col < L, ncc, NEG)

    for i in range(K):
        m = jnp.max(ncc, axis=1, keepdims=True)
        idx = jnp.min(jnp.where(ncc == m, col, LP), axis=1)
        idx_ref[0, i] = idx + base
        if i + 1 < K:
            ncc = jnp.where(col == idx[:, None], NEG, ncc)


def _fold_planes_body(g_ref, out_ref):
    g = g_ref[0]

    z_row = jnp.zeros((1, 64), jnp.float32)
    z_col = jnp.zeros((64, 1), jnp.float32)

    def shr_row(x):
        return jnp.concatenate([z_row, x[:63, :]], axis=0)

    def shr_col(x):
        return jnp.concatenate([z_col, x[:, :63]], axis=1)

    out_ref[0, 0, 0] = g[0] + shr_col(g[2]) + shr_row(g[6]) + shr_row(shr_col(g[8]))
    out_ref[0, 0, 1] = g[1] + shr_row(g[7])
    out_ref[0, 1, 0] = g[3] + shr_col(g[5])
    out_ref[0, 1, 1] = g[4]


def _sc_gather_body(table_hbm, idx_hbm, out_hbm, idx_v, rows_v, sem):
    wid = lax.axis_index("s") * 2 + lax.axis_index("c")
    pltpu.sync_copy(idx_hbm.at[wid], idx_v)
    nbuf = 4
    for g in range(NCH // nbuf):
        copies = [
            pltpu.async_copy(table_hbm.at[idx_v.at[g * nbuf + j]],
                             rows_v.at[j], sem)
            for j in range(nbuf)
        ]
        for c in copies:
            c.wait()
        pltpu.sync_copy(rows_v, out_hbm.at[wid, pl.ds(g * nbuf, nbuf)])


@jax.jit
def kernel(x, ref):
    B = x.shape[0]
    x_gray = (jnp.max(_maxpool3(x), axis=1, keepdims=True)
              - jnp.min(x, axis=1, keepdims=True))
    ref_gray = jnp.mean(ref, axis=1, keepdims=True)
    input_patch = _unfold(x_gray)
    ref_patch = _unfold(ref_gray)

    input_mu = jnp.mean(input_patch, axis=2, keepdims=True)
    ref_mu = jnp.mean(ref_patch, axis=2, keepdims=True)
    input_len = jnp.linalg.norm(input_patch, axis=1, keepdims=True)
    ref_len = jnp.linalg.norm(ref_patch, axis=1, keepdims=True)
    input_norm = (input_patch - input_mu) / input_len
    ref_norm = (ref_patch - ref_mu) / ref_len

    pad9 = [(0, 0), (0, 16 - 9), (0, LP - L)]
    inT = jnp.pad(input_norm, pad9).transpose(0, 2, 1)
    refn = jnp.pad(ref_norm, pad9)
    refpT = jnp.pad(ref_patch, pad9).transpose(0, 2, 1)

    nblk = LP // TL
    idx = pl.pallas_call(
        _ncc_topk_body,
        grid=(B, nblk),
        in_specs=[
            pl.BlockSpec((1, TL, 16), lambda b, r: (b, r, 0)),
            pl.BlockSpec((1, 16, LP), lambda b, r: (b, 0, 0)),
        ],
        out_specs=pl.BlockSpec((1, K, TL), lambda b, r: (b, 0, r)),
        out_shape=jax.ShapeDtypeStruct((B, K, LP), jnp.int32),
    )(inT, refn)

    table = jnp.pad(refpT, [(0, 0), (0, 0), (0, 128 - 16)])
    table = table.reshape(B * LP, 128)
    idx_w = idx.reshape(NW, NCH, 128)
    mesh = plsc.VectorSubcoreMesh(core_axis_name="c", subcore_axis_name="s",
                                  num_cores=2, num_subcores=16)
    sc_gather = functools.partial(
        pl.kernel,
        out_type=jax.ShapeDtypeStruct((NW, NCH, 128, 128), jnp.float32),
        mesh=mesh,
        scratch_types=[
            pltpu.VMEM((NCH, 128), jnp.int32),
            pltpu.VMEM((4, 128, 128), jnp.float32),
            pltpu.SemaphoreType.DMA,
        ],
    )(_sc_gather_body)
    gathered = sc_gather(table, idx_w)

    g = gathered.reshape(B, K, LP, 128)[:, :, :L, :9].transpose(0, 1, 3, 2)
    g = g.reshape(B * K, 9, HO, HO)
    g = jnp.pad(g, [(0, 0), (0, 0), (0, 1), (0, 1)])

    planes = pl.pallas_call(
        _fold_planes_body,
        grid=(B * K,),
        in_specs=[pl.BlockSpec((1, 9, 64, 64), lambda i: (i, 0, 0, 0))],
        out_specs=pl.BlockSpec((1, 2, 2, 64, 64), lambda i: (i, 0, 0, 0, 0)),
        out_shape=jax.ShapeDtypeStruct((B * K, 2, 2, 64, 64), jnp.float32),
    )(g)

    folds = planes.reshape(B, K, 2, 2, 64, 64).transpose(0, 1, 4, 2, 5, 3)
    folds = folds.reshape(B, K, P, P)
    return jnp.concatenate([x, folds], axis=1)


# speedup vs baseline: 37.6645x; 1.0949x over previous
import functools

import jax
import jax.numpy as jnp
from jax import lax
from jax.experimental import pallas as pl
from jax.experimental.pallas import tpu as pltpu, tpu_sc as plsc

K = 4
P = 128
HO = 63
L = HO * HO
LP = 4096
TL = 256
NEG = -3.0e38

NW = 32
NCH = 16


def _maxpool3(x):
    return jax.lax.reduce_window(x, -jnp.inf, jax.lax.max,
                                 (1, 1, 3, 3), (1, 1, 1, 1),
                                 [(0, 0), (0, 0), (1, 1), (1, 1)])


def _unfold(img):
    B, _, H, W = img.shape
    Ho = (H - 3) // 2 + 1
    Wo = (W - 3) // 2 + 1
    cols = []
    for di in range(3):
        for dj in range(3):
            cols.append(img[:, :, di:di + 2 * (Ho - 1) + 1:2,
                            dj:dj + 2 * (Wo - 1) + 1:2])
    patch = jnp.concatenate(cols, axis=1)
    return patch.reshape(B, 9, Ho * Wo)


def _ncc_topk_body(inT_ref, refn_ref, idx_ref):
    a = inT_ref[0]
    bm = refn_ref[0]
    base = pl.program_id(0) * LP

    ncc = jax.lax.dot_general(
        a, bm, (((1,), (0,)), ((), ())),
        preferred_element_type=jnp.float32,
        precision=jax.lax.Precision.DEFAULT)

    colf = jax.lax.broadcasted_iota(jnp.int32, (1, LP), 1).astype(jnp.float32)
    ncc = jnp.where(colf < float(L), ncc, NEG)

    for i in range(K):
        m = jnp.max(ncc, axis=1, keepdims=True)
        idxf = jnp.min(jnp.where(ncc == m, colf, float(LP)),
                       axis=1, keepdims=True)
        idx_ref[0, i] = idxf.astype(jnp.int32) + base
        if i + 1 < K:
            ncc = jnp.where(colf == idxf, NEG, ncc)


def _fold_planes_body(g_ref, out_ref):
    g = g_ref[0]

    z_row = jnp.zeros((1, 64), jnp.float32)
    z_col = jnp.zeros((64, 1), jnp.float32)

    def shr_row(x):
        return jnp.concatenate([z_row, x[:63, :]], axis=0)

    def shr_col(x):
        return jnp.concatenate([z_col, x[:, :63]], axis=1)

    out_ref[0, 0, 0] = g[0] + shr_col(g[2]) + shr_row(g[6]) + shr_row(shr_col(g[8]))
    out_ref[0, 0, 1] = g[1] + shr_row(g[7])
    out_ref[0, 1, 0] = g[3] + shr_col(g[5])
    out_ref[0, 1, 1] = g[4]


def _sc_gather_body(table_hbm, idx_hbm, out_hbm, idx_v, rows_v, sem):
    wid = lax.axis_index("s") * 2 + lax.axis_index("c")
    pltpu.sync_copy(idx_hbm.at[wid], idx_v)
    nbuf = 4
    for g in range(NCH // nbuf):
        copies = [
            pltpu.async_copy(table_hbm.at[idx_v.at[g * nbuf + j]],
                             rows_v.at[j], sem)
            for j in range(nbuf)
        ]
        for c in copies:
            c.wait()
        pltpu.sync_copy(rows_v, out_hbm.at[wid, pl.ds(g * nbuf, nbuf)])


@jax.jit
def kernel(x, ref):
    B = x.shape[0]
    x_gray = (jnp.max(_maxpool3(x), axis=1, keepdims=True)
              - jnp.min(x, axis=1, keepdims=True))
    ref_gray = jnp.mean(ref, axis=1, keepdims=True)
    input_patch = _unfold(x_gray)
    ref_patch = _unfold(ref_gray)

    input_mu = jnp.mean(input_patch, axis=2, keepdims=True)
    ref_mu = jnp.mean(ref_patch, axis=2, keepdims=True)
    input_len = jnp.linalg.norm(input_patch, axis=1, keepdims=True)
    ref_len = jnp.linalg.norm(ref_patch, axis=1, keepdims=True)
    input_norm = (input_patch - input_mu) / input_len
    ref_norm = (ref_patch - ref_mu) / ref_len

    pad9 = [(0, 0), (0, 16 - 9), (0, LP - L)]
    inT = jnp.pad(input_norm, pad9).transpose(0, 2, 1)
    refn = jnp.pad(ref_norm, pad9)
    refpT = jnp.pad(ref_patch, pad9).transpose(0, 2, 1)

    nblk = LP // TL
    idx = pl.pallas_call(
        _ncc_topk_body,
        grid=(B, nblk),
        in_specs=[
            pl.BlockSpec((1, TL, 16), lambda b, r: (b, r, 0)),
            pl.BlockSpec((1, 16, LP), lambda b, r: (b, 0, 0)),
        ],
        out_specs=pl.BlockSpec((1, K, TL, 1), lambda b, r: (b, 0, r, 0)),
        out_shape=jax.ShapeDtypeStruct((B, K, LP, 1), jnp.int32),
    )(inT, refn)

    table = jnp.pad(refpT, [(0, 0), (0, 0), (0, 128 - 16)])
    table = table.reshape(B * LP, 128)
    idx_w = idx.reshape(NW, NCH, 128)
    mesh = plsc.VectorSubcoreMesh(core_axis_name="c", subcore_axis_name="s",
                                  num_cores=2, num_subcores=16)
    sc_gather = functools.partial(
        pl.kernel,
        out_type=jax.ShapeDtypeStruct((NW, NCH, 128, 128), jnp.float32),
        mesh=mesh,
        scratch_types=[
            pltpu.VMEM((NCH, 128), jnp.int32),
            pltpu.VMEM((4, 128, 128), jnp.float32),
            pltpu.SemaphoreType.DMA,
        ],
    )(_sc_gather_body)
    gathered = sc_gather(table, idx_w)

    g = gathered.reshape(B, K, LP, 128)[:, :, :L, :9].transpose(0, 1, 3, 2)
    g = g.reshape(B * K, 9, HO, HO)
    g = jnp.pad(g, [(0, 0), (0, 0), (0, 1), (0, 1)])

    planes = pl.pallas_call(
        _fold_planes_body,
        grid=(B * K,),
        in_specs=[pl.BlockSpec((1, 9, 64, 64), lambda i: (i, 0, 0, 0))],
        out_specs=pl.BlockSpec((1, 2, 2, 64, 64), lambda i: (i, 0, 0, 0, 0)),
        out_shape=jax.ShapeDtypeStruct((B * K, 2, 2, 64, 64), jnp.float32),
    )(g)

    folds = planes.reshape(B, K, 2, 2, 64, 64).transpose(0, 1, 4, 2, 5, 3)
    folds = folds.reshape(B, K, P, P)
    return jnp.concatenate([x, folds], axis=1)


# speedup vs baseline: 39.1086x; 1.0383x over previous
import functools

import jax
import jax.numpy as jnp
import numpy as np
from jax import lax
from jax.experimental import pallas as pl
from jax.experimental.pallas import tpu as pltpu, tpu_sc as plsc

K = 4
P = 128
HO = 63
L = HO * HO
LP = 4096
TL = 256
NEG = -3.0e38

NW = 32
NCH = 16

_WALL = np.zeros((16, 16), np.float32)
for _p, _j, _grp in [(0, 0, 0), (1, 1, 0), (3, 2, 0), (4, 3, 0),
                     (2, 0, 1), (5, 2, 1),
                     (6, 0, 2), (7, 1, 2),
                     (8, 0, 3)]:
    _WALL[_p, 4 * _grp + _j] = 1.0


def _maxpool3(x):
    return jax.lax.reduce_window(x, -jnp.inf, jax.lax.max,
                                 (1, 1, 3, 3), (1, 1, 1, 1),
                                 [(0, 0), (0, 0), (1, 1), (1, 1)])


def _unfold(img):
    B, _, H, W = img.shape
    Ho = (H - 3) // 2 + 1
    Wo = (W - 3) // 2 + 1
    cols = []
    for di in range(3):
        for dj in range(3):
            cols.append(img[:, :, di:di + 2 * (Ho - 1) + 1:2,
                            dj:dj + 2 * (Wo - 1) + 1:2])
    patch = jnp.concatenate(cols, axis=1)
    return patch.reshape(B, 9, Ho * Wo)


def _ncc_topk_body(inP_ref, refn_ref, idx_ref):
    av = inP_ref[0]
    bm = refn_ref[0]
    base = pl.program_id(0) * LP

    ncc = jax.lax.dot_general(
        av, bm, (((0,), (0,)), ((), ())),
        preferred_element_type=jnp.float32,
        precision=jax.lax.Precision.DEFAULT)

    colf = jax.lax.broadcasted_iota(jnp.int32, (1, LP), 1).astype(jnp.float32)
    ncc = jnp.where(colf < float(L), ncc, NEG)

    for i in range(K):
        m = jnp.max(ncc, axis=1, keepdims=True)
        idxf = jnp.min(jnp.where(ncc == m, colf, float(LP)),
                       axis=1, keepdims=True)
        idx_ref[0, i] = idxf.astype(jnp.int32) + base
        if i + 1 < K:
            ncc = jnp.where(colf == idxf, NEG, ncc)


def _fold_body(g_ref, w_ref, out_ref):
    sel = jax.lax.dot_general(
        g_ref[0], w_ref[...], (((1,), (0,)), ((), ())),
        preferred_element_type=jnp.float32,
        precision=jax.lax.Precision.HIGHEST)
    planes = (
        sel[:, 0:4]
        + jnp.concatenate([jnp.zeros((1, 4), jnp.float32),
                           sel[:4095, 4:8]], axis=0)
        + jnp.concatenate([jnp.zeros((64, 4), jnp.float32),
                           sel[:4032, 8:12]], axis=0)
        + jnp.concatenate([jnp.zeros((65, 4), jnp.float32),
                           sel[:4031, 12:16]], axis=0)
    )
    out_ref[0] = planes


def _sc_gather_body(table_hbm, idx_hbm, out_hbm, idx_v, rows_v, sem):
    wid = lax.axis_index("s") * 2 + lax.axis_index("c")
    pltpu.sync_copy(idx_hbm.at[wid], idx_v)
    nbuf = 4
    for g in range(NCH // nbuf):
        copies = [
            pltpu.async_copy(table_hbm.at[idx_v.at[g * nbuf + j]],
                             rows_v.at[j], sem)
            for j in range(nbuf)
        ]
        for c in copies:
            c.wait()
        pltpu.sync_copy(rows_v, out_hbm.at[wid, pl.ds(g * nbuf, nbuf)])


@jax.jit
def kernel(x, ref):
    B = x.shape[0]
    x_gray = (jnp.max(_maxpool3(x), axis=1, keepdims=True)
              - jnp.min(x, axis=1, keepdims=True))
    ref_gray = jnp.mean(ref, axis=1, keepdims=True)
    input_patch = _unfold(x_gray)
    ref_patch = _unfold(ref_gray)

    input_mu = jnp.mean(input_patch, axis=2, keepdims=True)
    ref_mu = jnp.mean(ref_patch, axis=2, keepdims=True)
    input_len = jnp.linalg.norm(input_patch, axis=1, keepdims=True)
    ref_len = jnp.linalg.norm(ref_patch, axis=1, keepdims=True)
    input_norm = (input_patch - input_mu) / input_len
    ref_norm = (ref_patch - ref_mu) / ref_len

    pad9 = [(0, 0), (0, 16 - 9), (0, LP - L)]
    inP = jnp.pad(input_norm, pad9)
    refn = jnp.pad(ref_norm, pad9)
    refpT = jnp.pad(ref_patch, pad9).transpose(0, 2, 1)

    nblk = LP // TL
    idx = pl.pallas_call(
        _ncc_topk_body,
        grid=(B, nblk),
        in_specs=[
            pl.BlockSpec((1, 16, TL), lambda b, r: (b, 0, r)),
            pl.BlockSpec((1, 16, LP), lambda b, r: (b, 0, 0)),
        ],
        out_specs=pl.BlockSpec((1, K, TL, 1), lambda b, r: (b, 0, r, 0)),
        out_shape=jax.ShapeDtypeStruct((B, K, LP, 1), jnp.int32),
    )(inP, refn)

    idxs = idx[:, :, :L, 0].reshape(B, K, HO, HO)
    idx_f = jnp.pad(idxs, [(0, 0), (0, 0), (0, 1), (0, 1)])
    zrow = (jnp.arange(B, dtype=jnp.int32) * LP + L).reshape(B, 1, 1, 1)
    valid = jnp.pad(jnp.ones((HO, HO), jnp.bool_), [(0, 1), (0, 1)])
    idx_f = jnp.where(valid, idx_f, zrow)

    table = refpT.reshape(B * LP, 16)
    idx_w = idx_f.reshape(NW, NCH, 128)
    mesh = plsc.VectorSubcoreMesh(core_axis_name="c", subcore_axis_name="s",
                                  num_cores=2, num_subcores=16)
    sc_gather = functools.partial(
        pl.kernel,
        out_type=jax.ShapeDtypeStruct((NW, NCH, 128, 16), jnp.float32),
        compiler_params=pltpu.CompilerParams(use_tc_tiling_on_sc=False),
        mesh=mesh,
        scratch_types=[
            pltpu.VMEM((NCH, 128), jnp.int32),
            pltpu.VMEM((4, 128, 16), jnp.float32),
            pltpu.SemaphoreType.DMA,
        ],
    )(_sc_gather_body)
    gathered = sc_gather(table, idx_w)

    planes = pl.pallas_call(
        _fold_body,
        grid=(B * K,),
        in_specs=[
            pl.BlockSpec((1, 64 * 64, 16), lambda i: (i, 0, 0)),
            pl.BlockSpec((16, 16), lambda i: (0, 0)),
        ],
        out_specs=pl.BlockSpec((1, 64 * 64, 4), lambda i: (i, 0, 0)),
        out_shape=jax.ShapeDtypeStruct((B * K, 64 * 64, 4), jnp.float32),
    )(gathered.reshape(B * K, 64 * 64, 16), jnp.asarray(_WALL))

    folds = planes.reshape(B, K, 64, 64, 2, 2).transpose(0, 1, 2, 4, 3, 5)
    folds = folds.reshape(B, K, P, P)
    return jnp.concatenate([x, folds], axis=1)


# speedup vs baseline: 39.8949x; 1.0201x over previous
import functools

import jax
import jax.numpy as jnp
import numpy as np
from jax import lax
from jax.experimental import pallas as pl
from jax.experimental.pallas import tpu as pltpu, tpu_sc as plsc

K = 4
P = 128
HO = 63
L = HO * HO
LP = 4096
TL = 512
NEG = -3.0e38

NW = 32
NCH = 16

_WALL = np.zeros((16, 16), np.float32)
for _p, _j, _grp in [(0, 0, 0), (1, 1, 0), (3, 2, 0), (4, 3, 0),
                     (2, 0, 1), (5, 2, 1),
                     (6, 0, 2), (7, 1, 2),
                     (8, 0, 3)]:
    _WALL[_p, 4 * _grp + _j] = 1.0


def _maxpool3(x):
    return jax.lax.reduce_window(x, -jnp.inf, jax.lax.max,
                                 (1, 1, 3, 3), (1, 1, 1, 1),
                                 [(0, 0), (0, 0), (1, 1), (1, 1)])


def _unfold(img):
    B, _, H, W = img.shape
    Ho = (H - 3) // 2 + 1
    Wo = (W - 3) // 2 + 1
    cols = []
    for di in range(3):
        for dj in range(3):
            cols.append(img[:, :, di:di + 2 * (Ho - 1) + 1:2,
                            dj:dj + 2 * (Wo - 1) + 1:2])
    patch = jnp.concatenate(cols, axis=1)
    return patch.reshape(B, 9, Ho * Wo)


def _ncc_topk_body(inP_ref, refn_ref, idx_ref):
    av = inP_ref[0]
    bm = refn_ref[0]
    base = pl.program_id(0) * LP

    ncc = jax.lax.dot_general(
        av, bm, (((0,), (0,)), ((), ())),
        preferred_element_type=jnp.float32,
        precision=jax.lax.Precision.DEFAULT)

    colf = jax.lax.broadcasted_iota(jnp.int32, (1, LP), 1).astype(jnp.float32)
    ncc = jnp.where(colf < float(L), ncc, NEG)

    for i in range(K):
        m = jnp.max(ncc, axis=1, keepdims=True)
        idxf = jnp.min(jnp.where(ncc == m, colf, float(LP)),
                       axis=1, keepdims=True)
        idx_ref[0, i] = idxf.astype(jnp.int32) + base
        if i + 1 < K:
            ncc = jnp.where(colf == idxf, NEG, ncc)


def _fold_body(g_ref, w_ref, out_ref):
    sel = jax.lax.dot_general(
        g_ref[0], w_ref[...], (((1,), (0,)), ((), ())),
        preferred_element_type=jnp.float32,
        precision=jax.lax.Precision.HIGHEST)
    planes = (
        sel[:, 0:4]
        + jnp.concatenate([jnp.zeros((1, 4), jnp.float32),
                           sel[:4095, 4:8]], axis=0)
        + jnp.concatenate([jnp.zeros((64, 4), jnp.float32),
                           sel[:4032, 8:12]], axis=0)
        + jnp.concatenate([jnp.zeros((65, 4), jnp.float32),
                           sel[:4031, 12:16]], axis=0)
    )
    out_ref[0] = planes


def _sc_gather_body(table_hbm, idx_hbm, out_hbm, idx_v, rows_v, sem):
    wid = lax.axis_index("s") * 2 + lax.axis_index("c")
    pltpu.sync_copy(idx_hbm.at[wid], idx_v)
    nbuf = 4
    for g in range(NCH // nbuf):
        copies = [
            pltpu.async_copy(table_hbm.at[idx_v.at[g * nbuf + j]],
                             rows_v.at[j], sem)
            for j in range(nbuf)
        ]
        for c in copies:
            c.wait()
        pltpu.sync_copy(rows_v, out_hbm.at[wid, pl.ds(g * nbuf, nbuf)])


@jax.jit
def kernel(x, ref):
    B = x.shape[0]
    x_gray = (jnp.max(_maxpool3(x), axis=1, keepdims=True)
              - jnp.min(x, axis=1, keepdims=True))
    ref_gray = jnp.mean(ref, axis=1, keepdims=True)
    input_patch = _unfold(x_gray)
    ref_patch = _unfold(ref_gray)

    input_mu = jnp.mean(input_patch, axis=2, keepdims=True)
    ref_mu = jnp.mean(ref_patch, axis=2, keepdims=True)
    input_len = jnp.linalg.norm(input_patch, axis=1, keepdims=True)
    ref_len = jnp.linalg.norm(ref_patch, axis=1, keepdims=True)
    input_norm = (input_patch - input_mu) / input_len
    ref_norm = (ref_patch - ref_mu) / ref_len

    pad9 = [(0, 0), (0, 16 - 9), (0, LP - L)]
    inP = jnp.pad(input_norm, pad9)
    refn = jnp.pad(ref_norm, pad9)
    refpT = jnp.pad(ref_patch, pad9).transpose(0, 2, 1)

    nblk = LP // TL
    idx = pl.pallas_call(
        _ncc_topk_body,
        grid=(B, nblk),
        in_specs=[
            pl.BlockSpec((1, 16, TL), lambda b, r: (b, 0, r)),
            pl.BlockSpec((1, 16, LP), lambda b, r: (b, 0, 0)),
        ],
        out_specs=pl.BlockSpec((1, K, TL, 1), lambda b, r: (b, 0, r, 0)),
        out_shape=jax.ShapeDtypeStruct((B, K, LP, 1), jnp.int32),
    )(inP, refn)

    idxs = idx[:, :, :L, 0].reshape(B, K, HO, HO)
    idx_f = jnp.pad(idxs, [(0, 0), (0, 0), (0, 1), (0, 1)])
    zrow = (jnp.arange(B, dtype=jnp.int32) * LP + L).reshape(B, 1, 1, 1)
    valid = jnp.pad(jnp.ones((HO, HO), jnp.bool_), [(0, 1), (0, 1)])
    idx_f = jnp.where(valid, idx_f, zrow)

    table = refpT.reshape(B * LP, 16)
    idx_w = idx_f.reshape(NW, NCH, 128)
    mesh = plsc.VectorSubcoreMesh(core_axis_name="c", subcore_axis_name="s",
                                  num_cores=2, num_subcores=16)
    sc_gather = functools.partial(
        pl.kernel,
        out_type=jax.ShapeDtypeStruct((NW, NCH, 128, 16), jnp.float32),
        compiler_params=pltpu.CompilerParams(use_tc_tiling_on_sc=False),
        mesh=mesh,
        scratch_types=[
            pltpu.VMEM((NCH, 128), jnp.int32),
            pltpu.VMEM((4, 128, 16), jnp.float32),
            pltpu.SemaphoreType.DMA,
        ],
    )(_sc_gather_body)
    gathered = sc_gather(table, idx_w)

    planes = pl.pallas_call(
        _fold_body,
        grid=(B * K,),
        in_specs=[
            pl.BlockSpec((1, 64 * 64, 16), lambda i: (i, 0, 0)),
            pl.BlockSpec((16, 16), lambda i: (0, 0)),
        ],
        out_specs=pl.BlockSpec((1, 64 * 64, 4), lambda i: (i, 0, 0)),
        out_shape=jax.ShapeDtypeStruct((B * K, 64 * 64, 4), jnp.float32),
    )(gathered.reshape(B * K, 64 * 64, 16), jnp.asarray(_WALL))

    folds = planes.reshape(B, K, 64, 64, 2, 2).transpose(0, 1, 2, 4, 3, 5)
    folds = folds.reshape(B, K, P, P)
    return jnp.concatenate([x, folds], axis=1)


# speedup vs baseline: 41.0098x; 1.0279x over previous
import functools

import jax
import jax.numpy as jnp
import numpy as np
from jax import lax
from jax.experimental import pallas as pl
from jax.experimental.pallas import tpu as pltpu, tpu_sc as plsc

K = 4
P = 128
HO = 63
L = HO * HO
LP = 4096
TL = 512
NEG = -3.0e38

NW = 32
NCH = 16

_WALL = np.zeros((16, 16), np.float32)
for _p, _j, _grp in [(0, 0, 0), (1, 1, 0), (3, 2, 0), (4, 3, 0),
                     (2, 0, 1), (5, 2, 1),
                     (6, 0, 2), (7, 1, 2),
                     (8, 0, 3)]:
    _WALL[_p, 4 * _grp + _j] = 1.0


def _maxpool3(x):
    return jax.lax.reduce_window(x, -jnp.inf, jax.lax.max,
                                 (1, 1, 3, 3), (1, 1, 1, 1),
                                 [(0, 0), (0, 0), (1, 1), (1, 1)])


def _unfold(img):
    B, _, H, W = img.shape
    Ho = (H - 3) // 2 + 1
    Wo = (W - 3) // 2 + 1
    cols = []
    for di in range(3):
        for dj in range(3):
            cols.append(img[:, :, di:di + 2 * (Ho - 1) + 1:2,
                            dj:dj + 2 * (Wo - 1) + 1:2])
    patch = jnp.concatenate(cols, axis=1)
    return patch.reshape(B, 9, Ho * Wo)


def _ncc_topk_body(inP_ref, refn_ref, idx_ref):
    av = inP_ref[0]
    bm = refn_ref[0]
    base = pl.program_id(0) * LP

    ncc = jax.lax.dot_general(
        av, bm, (((0,), (0,)), ((), ())),
        preferred_element_type=jnp.float32,
        precision=jax.lax.Precision.DEFAULT)

    colf = jax.lax.broadcasted_iota(jnp.int32, (1, LP), 1).astype(jnp.float32)

    for i in range(K):
        m = jnp.max(ncc, axis=1, keepdims=True)
        idxf = jnp.min(jnp.where(ncc == m, colf, float(LP)),
                       axis=1, keepdims=True)
        idx_ref[0, i] = idxf.astype(jnp.int32) + base
        if i + 1 < K:
            ncc = jnp.where(colf == idxf, NEG, ncc)


def _fold_body(g_ref, w_ref, out_ref):
    sel = jax.lax.dot_general(
        g_ref[0], w_ref[...], (((1,), (0,)), ((), ())),
        preferred_element_type=jnp.float32,
        precision=jax.lax.Precision.HIGHEST)
    planes = (
        sel[:, 0:4]
        + jnp.concatenate([jnp.zeros((1, 4), jnp.float32),
                           sel[:4095, 4:8]], axis=0)
        + jnp.concatenate([jnp.zeros((64, 4), jnp.float32),
                           sel[:4032, 8:12]], axis=0)
        + jnp.concatenate([jnp.zeros((65, 4), jnp.float32),
                           sel[:4031, 12:16]], axis=0)
    )
    out_ref[0] = planes


def _sc_gather_body(table_hbm, idx_hbm, out_hbm, idx_v, rows_v, sem):
    wid = lax.axis_index("s") * 2 + lax.axis_index("c")
    pltpu.sync_copy(idx_hbm.at[wid], idx_v)
    nbuf = 4
    for g in range(NCH // nbuf):
        copies = [
            pltpu.async_copy(table_hbm.at[idx_v.at[g * nbuf + j]],
                             rows_v.at[j], sem)
            for j in range(nbuf)
        ]
        for c in copies:
            c.wait()
        pltpu.sync_copy(rows_v, out_hbm.at[wid, pl.ds(g * nbuf, nbuf)])


@jax.jit
def kernel(x, ref):
    B = x.shape[0]
    x_gray = (jnp.max(_maxpool3(x), axis=1, keepdims=True)
              - jnp.min(x, axis=1, keepdims=True))
    ref_gray = jnp.mean(ref, axis=1, keepdims=True)
    input_patch = _unfold(x_gray)
    ref_patch = _unfold(ref_gray)

    input_mu = jnp.mean(input_patch, axis=2, keepdims=True)
    ref_mu = jnp.mean(ref_patch, axis=2, keepdims=True)
    input_len = jnp.linalg.norm(input_patch, axis=1, keepdims=True)
    ref_len = jnp.linalg.norm(ref_patch, axis=1, keepdims=True)
    input_norm = (input_patch - input_mu) / input_len
    ref_norm = (ref_patch - ref_mu) / ref_len

    pad9 = [(0, 0), (0, 16 - 9), (0, LP - L)]
    inP = jnp.pad(input_norm, pad9).at[:, 9, :].set(1.0)
    refn = jnp.pad(ref_norm, pad9).at[:, 9, L:].set(NEG)
    refpT = jnp.pad(ref_patch, pad9).transpose(0, 2, 1)

    nblk = LP // TL
    idx = pl.pallas_call(
        _ncc_topk_body,
        grid=(B, nblk),
        in_specs=[
            pl.BlockSpec((1, 16, TL), lambda b, r: (b, 0, r)),
            pl.BlockSpec((1, 16, LP), lambda b, r: (b, 0, 0)),
        ],
        out_specs=pl.BlockSpec((1, K, TL, 1), lambda b, r: (b, 0, r, 0)),
        out_shape=jax.ShapeDtypeStruct((B, K, LP, 1), jnp.int32),
    )(inP, refn)

    idxs = idx[:, :, :L, 0].reshape(B, K, HO, HO)
    idx_f = jnp.pad(idxs, [(0, 0), (0, 0), (0, 1), (0, 1)])
    zrow = (jnp.arange(B, dtype=jnp.int32) * LP + L).reshape(B, 1, 1, 1)
    valid = jnp.pad(jnp.ones((HO, HO), jnp.bool_), [(0, 1), (0, 1)])
    idx_f = jnp.where(valid, idx_f, zrow)

    table = refpT.reshape(B * LP, 16)
    idx_w = idx_f.reshape(NW, NCH, 128)
    mesh = plsc.VectorSubcoreMesh(core_axis_name="c", subcore_axis_name="s",
                                  num_cores=2, num_subcores=16)
    sc_gather = functools.partial(
        pl.kernel,
        out_type=jax.ShapeDtypeStruct((NW, NCH, 128, 16), jnp.float32),
        compiler_params=pltpu.CompilerParams(use_tc_tiling_on_sc=False),
        mesh=mesh,
        scratch_types=[
            pltpu.VMEM((NCH, 128), jnp.int32),
            pltpu.VMEM((4, 128, 16), jnp.float32),
            pltpu.SemaphoreType.DMA,
        ],
    )(_sc_gather_body)
    gathered = sc_gather(table, idx_w)

    planes = pl.pallas_call(
        _fold_body,
        grid=(B * K,),
        in_specs=[
            pl.BlockSpec((1, 64 * 64, 16), lambda i: (i, 0, 0)),
            pl.BlockSpec((16, 16), lambda i: (0, 0)),
        ],
        out_specs=pl.BlockSpec((1, 64 * 64, 4), lambda i: (i, 0, 0)),
        out_shape=jax.ShapeDtypeStruct((B * K, 64 * 64, 4), jnp.float32),
    )(gathered.reshape(B * K, 64 * 64, 16), jnp.asarray(_WALL))

    folds = planes.reshape(B, K, 64, 64, 2, 2).transpose(0, 1, 2, 4, 3, 5)
    folds = folds.reshape(B, K, P, P)
    return jnp.concatenate([x, folds], axis=1)


# speedup vs baseline: 41.4746x; 1.0113x over previous
import functools

import jax
import jax.numpy as jnp
import numpy as np
from jax import lax
from jax.experimental import pallas as pl
from jax.experimental.pallas import tpu as pltpu, tpu_sc as plsc

K = 4
P = 128
HO = 63
L = HO * HO
LP = 4096
TL = 1024
NEG = -3.0e38

NW = 32
NCH = 16

_WALL = np.zeros((16, 16), np.float32)
for _p, _j, _grp in [(0, 0, 0), (1, 1, 0), (3, 2, 0), (4, 3, 0),
                     (2, 0, 1), (5, 2, 1),
                     (6, 0, 2), (7, 1, 2),
                     (8, 0, 3)]:
    _WALL[_p, 4 * _grp + _j] = 1.0


def _maxpool3(x):
    return jax.lax.reduce_window(x, -jnp.inf, jax.lax.max,
                                 (1, 1, 3, 3), (1, 1, 1, 1),
                                 [(0, 0), (0, 0), (1, 1), (1, 1)])


def _unfold(img):
    B, _, H, W = img.shape
    Ho = (H - 3) // 2 + 1
    Wo = (W - 3) // 2 + 1
    cols = []
    for di in range(3):
        for dj in range(3):
            cols.append(img[:, :, di:di + 2 * (Ho - 1) + 1:2,
                            dj:dj + 2 * (Wo - 1) + 1:2])
    patch = jnp.concatenate(cols, axis=1)
    return patch.reshape(B, 9, Ho * Wo)


def _ncc_topk_body(inP_ref, refn_ref, idx_ref):
    av = inP_ref[0]
    bm = refn_ref[0]
    base = pl.program_id(0) * LP

    ncc = jax.lax.dot_general(
        av, bm, (((0,), (0,)), ((), ())),
        preferred_element_type=jnp.float32,
        precision=jax.lax.Precision.DEFAULT)

    colf = jax.lax.broadcasted_iota(jnp.int32, (1, LP), 1).astype(jnp.float32)

    for i in range(K):
        m = jnp.max(ncc, axis=1, keepdims=True)
        idxf = jnp.min(jnp.where(ncc == m, colf, float(LP)),
                       axis=1, keepdims=True)
        idx_ref[0, i] = idxf.astype(jnp.int32) + base
        if i + 1 < K:
            ncc = jnp.where(colf == idxf, NEG, ncc)


def _fold_body(g_ref, w_ref, out_ref):
    sel = jax.lax.dot_general(
        g_ref[0], w_ref[...], (((1,), (0,)), ((), ())),
        preferred_element_type=jnp.float32,
        precision=jax.lax.Precision.HIGHEST)
    planes = (
        sel[:, 0:4]
        + jnp.concatenate([jnp.zeros((1, 4), jnp.float32),
                           sel[:4095, 4:8]], axis=0)
        + jnp.concatenate([jnp.zeros((64, 4), jnp.float32),
                           sel[:4032, 8:12]], axis=0)
        + jnp.concatenate([jnp.zeros((65, 4), jnp.float32),
                           sel[:4031, 12:16]], axis=0)
    )
    out_ref[0] = planes


def _sc_gather_body(table_hbm, idx_hbm, out_hbm, idx_v, rows_v, sem):
    wid = lax.axis_index("s") * 2 + lax.axis_index("c")
    pltpu.sync_copy(idx_hbm.at[wid], idx_v)
    nbuf = 4
    for g in range(NCH // nbuf):
        copies = [
            pltpu.async_copy(table_hbm.at[idx_v.at[g * nbuf + j]],
                             rows_v.at[j], sem)
            for j in range(nbuf)
        ]
        for c in copies:
            c.wait()
        pltpu.sync_copy(rows_v, out_hbm.at[wid, pl.ds(g * nbuf, nbuf)])


@jax.jit
def kernel(x, ref):
    B = x.shape[0]
    x_gray = (jnp.max(_maxpool3(x), axis=1, keepdims=True)
              - jnp.min(x, axis=1, keepdims=True))
    ref_gray = jnp.mean(ref, axis=1, keepdims=True)
    input_patch = _unfold(x_gray)
    ref_patch = _unfold(ref_gray)

    input_mu = jnp.mean(input_patch, axis=2, keepdims=True)
    ref_mu = jnp.mean(ref_patch, axis=2, keepdims=True)
    input_len = jnp.linalg.norm(input_patch, axis=1, keepdims=True)
    ref_len = jnp.linalg.norm(ref_patch, axis=1, keepdims=True)
    input_norm = (input_patch - input_mu) / input_len
    ref_norm = (ref_patch - ref_mu) / ref_len

    pad9 = [(0, 0), (0, 16 - 9), (0, LP - L)]
    inP = jnp.pad(input_norm, pad9).at[:, 9, :].set(1.0)
    refn = jnp.pad(ref_norm, pad9).at[:, 9, L:].set(NEG)
    refpT = jnp.pad(ref_patch, pad9).transpose(0, 2, 1)

    nblk = LP // TL
    idx = pl.pallas_call(
        _ncc_topk_body,
        grid=(B, nblk),
        in_specs=[
            pl.BlockSpec((1, 16, TL), lambda b, r: (b, 0, r)),
            pl.BlockSpec((1, 16, LP), lambda b, r: (b, 0, 0)),
        ],
        out_specs=pl.BlockSpec((1, K, TL, 1), lambda b, r: (b, 0, r, 0)),
        out_shape=jax.ShapeDtypeStruct((B, K, LP, 1), jnp.int32),
    )(inP, refn)

    idxs = idx[:, :, :L, 0].reshape(B, K, HO, HO)
    idx_f = jnp.pad(idxs, [(0, 0), (0, 0), (0, 1), (0, 1)])
    zrow = (jnp.arange(B, dtype=jnp.int32) * LP + L).reshape(B, 1, 1, 1)
    valid = jnp.pad(jnp.ones((HO, HO), jnp.bool_), [(0, 1), (0, 1)])
    idx_f = jnp.where(valid, idx_f, zrow)

    table = refpT.reshape(B * LP, 16)
    idx_w = idx_f.reshape(NW, NCH, 128)
    mesh = plsc.VectorSubcoreMesh(core_axis_name="c", subcore_axis_name="s",
                                  num_cores=2, num_subcores=16)
    sc_gather = functools.partial(
        pl.kernel,
        out_type=jax.ShapeDtypeStruct((NW, NCH, 128, 16), jnp.float32),
        compiler_params=pltpu.CompilerParams(use_tc_tiling_on_sc=False),
        mesh=mesh,
        scratch_types=[
            pltpu.VMEM((NCH, 128), jnp.int32),
            pltpu.VMEM((4, 128, 16), jnp.float32),
            pltpu.SemaphoreType.DMA,
        ],
    )(_sc_gather_body)
    gathered = sc_gather(table, idx_w)

    planes = pl.pallas_call(
        _fold_body,
        grid=(B * K,),
        in_specs=[
            pl.BlockSpec((1, 64 * 64, 16), lambda i: (i, 0, 0)),
            pl.BlockSpec((16, 16), lambda i: (0, 0)),
        ],
        out_specs=pl.BlockSpec((1, 64 * 64, 4), lambda i: (i, 0, 0)),
        out_shape=jax.ShapeDtypeStruct((B * K, 64 * 64, 4), jnp.float32),
    )(gathered.reshape(B * K, 64 * 64, 16), jnp.asarray(_WALL))

    folds = planes.reshape(B, K, 64, 64, 2, 2).transpose(0, 1, 2, 4, 3, 5)
    folds = folds.reshape(B, K, P, P)
    return jnp.concatenate([x, folds], axis=1)
